# baseline (speedup 1.0000x reference)
import functools

import numpy as np

import jax
import jax.numpy as jnp
from jax import lax
from jax.experimental import pallas as pl
from jax.experimental.pallas import tpu as pltpu
from jax.experimental.pallas import tpu_sc as plsc

N = 10000
NP = 10240
D = 128
E = 320000
NL = 4
NC, NS = 2, 16
NW = NC * NS
EPW = E // NW
CH = 128
K = 8
NCH = NP // CH
RPT = NP // NS
NEG = -1e30

_f32 = jnp.float32

_PERM = np.concatenate([np.arange(0, D, 2), np.arange(1, D, 2)])
_INV = np.argsort(_PERM)


def _sds(shape, dtype=_f32):
    return jax.ShapeDtypeStruct(shape, dtype)


def _tc_edge_body(ea_ref, we_ref, ae_ref, aed_ref, mae_ref):
    rv = jnp.sum(we_ref[...] * ae_ref[...][:, None, :], axis=-1,
                 keepdims=True)
    for l in range(NL):
        acc = None
        for j in range(3):
            coef = rv[l, j:j + 1, :]
            t = ea_ref[j] * coef
            acc = t if acc is None else acc + t
        aed_ref[l] = acc
        mae_ref[l:l + 1, :] = jnp.max(acc, axis=0, keepdims=True)


def _tc_edge(eaT, We, att_e):
    return pl.pallas_call(
        _tc_edge_body,
        out_shape=(_sds((NL, E // D, D)), _sds((NL, D))),
    )(eaT, We, att_e)


_BR = 2048


def _attn_tail(h, as_ref, ad_ref, mae_ref, s_ref, d_ref, m_ref, sm_ref):
    s = jnp.sum(h * as_ref[...][None, :], axis=-1, keepdims=True)
    d = jnp.sum(h * ad_ref[...][None, :], axis=-1, keepdims=True)
    s_ref[...] = s
    d_ref[...] = d
    ms = jnp.max(s)
    md = jnp.max(d)

    @pl.when(pl.program_id(0) == 0)
    def _():
        sm_ref[0] = ms
        sm_ref[1] = md
        m_ref[...] = jnp.zeros((1, D), _f32)

    @pl.when(pl.program_id(0) > 0)
    def _():
        sm_ref[0] = jnp.maximum(sm_ref[0], ms)
        sm_ref[1] = jnp.maximum(sm_ref[1], md)

    @pl.when(pl.program_id(0) == NP // _BR - 1)
    def _():
        M = jnp.maximum(sm_ref[0] + sm_ref[1] + jnp.max(mae_ref[...]), 0.0)
        m_ref[...] = jnp.broadcast_to(M, (1, D))


def _tc_in_body(x_ref, w_ref, as_ref, ad_ref, mae_ref,
                h_ref, s_ref, d_ref, m_ref, sm_ref):
    h = jnp.dot(x_ref[...], w_ref[...], preferred_element_type=_f32)
    h_ref[...] = h
    _attn_tail(h, as_ref, ad_ref, mae_ref, s_ref, d_ref, m_ref, sm_ref)


_NODE_OUT_SPECS = [
    pl.BlockSpec((_BR, D), lambda i: (i, 0)),
    pl.BlockSpec((_BR, 1), lambda i: (i, 0)),
    pl.BlockSpec((_BR, 1), lambda i: (i, 0)),
    pl.BlockSpec((1, D), lambda i: (0, 0)),
]
_NODE_OUT_SHAPE = (_sds((NP, D)), _sds((NP, 1)),
                   _sds((NP, 1)), _sds((1, D)))


def _tc_in(xp, W0, as0, ad0, mae0):
    return pl.pallas_call(
        _tc_in_body,
        grid=(NP // _BR,),
        in_specs=[
            pl.BlockSpec((_BR, D), lambda i: (i, 0)),
            pl.BlockSpec((D, D), lambda i: (0, 0)),
            pl.BlockSpec((D,), lambda i: (0,)),
            pl.BlockSpec((D,), lambda i: (0,)),
            pl.BlockSpec((D,), lambda i: (0,)),
        ],
        out_specs=_NODE_OUT_SPECS,
        out_shape=_NODE_OUT_SHAPE,
        scratch_shapes=[pltpu.SMEM((2,), _f32)],
    )(xp, W0, as0, ad0, mae0)


def _act(acc_ref, den_ref, b_ref):
    den = den_ref[0] + den_ref[1]
    act = (acc_ref[0] + acc_ref[1]) * (1.0 / (den + 1e-16)) \
        + b_ref[...][None, :]
    return jnp.maximum(act, 0.0)


def _tc_mid_body(acc_ref, den_ref, b_ref, w_ref, as_ref, ad_ref, mae_ref,
                 h_ref, s_ref, d_ref, m_ref, sm_ref):
    act = _act(acc_ref, den_ref, b_ref)
    h = jnp.dot(act, w_ref[...], preferred_element_type=_f32)
    h_ref[...] = h
    _attn_tail(h, as_ref, ad_ref, mae_ref, s_ref, d_ref, m_ref, sm_ref)


def _tc_mid(acc, den, bl, Wl, asl, adl, mael):
    return pl.pallas_call(
        _tc_mid_body,
        grid=(NP // _BR,),
        in_specs=[
            pl.BlockSpec((NC, _BR, D), lambda i: (0, i, 0)),
            pl.BlockSpec((NC, _BR, 1), lambda i: (0, i, 0)),
            pl.BlockSpec((D,), lambda i: (0,)),
            pl.BlockSpec((D, D), lambda i: (0, 0)),
            pl.BlockSpec((D,), lambda i: (0,)),
            pl.BlockSpec((D,), lambda i: (0,)),
            pl.BlockSpec((D,), lambda i: (0,)),
        ],
        out_specs=_NODE_OUT_SPECS,
        out_shape=_NODE_OUT_SHAPE,
        scratch_shapes=[pltpu.SMEM((2,), _f32)],
    )(acc, den, bl, Wl, asl, adl, mael)


def _tc_fin_body(acc_ref, den_ref, b_ref, g_ref, be_ref, o_ref):
    act = _act(acc_ref, den_ref, b_ref)
    mu = jnp.mean(act, axis=-1, keepdims=True)
    xc = act - mu
    var = jnp.mean(xc * xc, axis=-1, keepdims=True)
    o_ref[...] = g_ref[...][None, :] * xc * lax.rsqrt(var + 1e-5) \
        + be_ref[...][None, :]


def _tc_fin(acc, den, bl, gamma, beta):
    return pl.pallas_call(
        _tc_fin_body,
        grid=(NP // _BR,),
        in_specs=[
            pl.BlockSpec((NC, _BR, D), lambda i: (0, i, 0)),
            pl.BlockSpec((NC, _BR, 1), lambda i: (0, i, 0)),
            pl.BlockSpec((D,), lambda i: (0,)),
            pl.BlockSpec((D,), lambda i: (0,)),
            pl.BlockSpec((D,), lambda i: (0,)),
        ],
        out_specs=pl.BlockSpec((_BR, D), lambda i: (i, 0)),
        out_shape=_sds((NP, D)),
    )(acc, den, bl, gamma, beta)


DW = D // 2


def _sc_body(h_hbm, as_hbm, ad_hbm, m_hbm, src_hbm, dst_hbm, ae_hbm,
             acc_hbm, d0_hbm, d1_hbm,
             mv, sb0, sb1, db0, db1, aec0, aec1,
             asv0, asv1, adv0, adv1, exc0, exc1,
             rbp0, rbp1,
             accs, dens, asrs, adss,
             srow0, srow1, ssc0, ssc1,
             ssmA0, ssmA1, ssmD0, ssmD1, sden0, sden1):
    cid = lax.axis_index("c")
    sid = lax.axis_index("s")
    wid = cid * NS + sid
    rb = (rbp0, rbp1)
    srow = (srow0, srow1)
    ssc = (ssc0, ssc1)
    asv = (asv0, asv1)
    adv = (adv0, adv1)
    exc = (exc0, exc1)
    ssmA = (ssmA0, ssmA1)
    ssmD = (ssmD0, ssmD1)
    sden = (sden0, sden1)

    pltpu.sync_copy(m_hbm, mv)

    @pl.when(sid == 0)
    def _():
        pltpu.sync_copy(as_hbm, asrs)
        pltpu.sync_copy(ad_hbm, adss)

    zero16 = jnp.zeros((16,), _f32)

    for kk in range(CH // 16):
        exc0[pl.ds(kk * 16, 16)] = zero16

    def _z2(r, _):
        for kk in range(D // 16):
            rbp0[r, pl.ds(kk * 16, 16)] = zero16
        return 0
    lax.fori_loop(0, CH, _z2, 0)

    for t in range(RPT // CH):
        pltpu.sync_copy(rbp0, accs.at[pl.ds(sid * RPT + t * CH, CH)])
        pltpu.sync_copy(exc0, dens.at[pl.ds(sid * RPT + t * CH, CH)])
    plsc.subcore_barrier()

    Mv = mv[pl.ds(0, 16)]

    def _scale(exr, rbr):
        def body(r, _):
            eb = plsc.load_gather(exr, [jnp.full((16,), r, jnp.int32)])
            for kk in range(D // 16):
                sl = pl.ds(kk * 16, 16)
                rbr[r, sl] = rbr[r, sl] * eb
            return 0
        lax.fori_loop(0, CH, body, 0, unroll=8)

    def _group(g, sg, dg, aeg):
        pltpu.sync_copy(src_hbm.at[wid, pl.ds(g * K, K)], sg)
        pltpu.sync_copy(dst_hbm.at[wid, pl.ds(g * K, K)], dg)
        pltpu.sync_copy(ae_hbm.at[wid, pl.ds(g * K, K)], aeg)
        def _gat2(i2, jb):
            h1 = CH // 2
            return (pltpu.async_copy(h_hbm.at[sg.at[i2, pl.ds(0, h1)]],
                                     rb[jb].at[pl.ds(0, h1)],
                                     srow[jb]),
                    pltpu.async_copy(h_hbm.at[sg.at[i2, pl.ds(h1, h1)]],
                                     rb[jb].at[pl.ds(h1, h1)],
                                     srow[jb]))

        gat = _gat2(0, 0)
        smA = pltpu.async_copy(asrs.at[sg.at[0]], asv[0], ssmA[0])
        smD = pltpu.async_copy(adss.at[dg.at[0]], adv[0], ssmD[0])
        sc_prev = None
        den_prev = [None, None]
        for i in range(K):
            ib = i & 1
            smA.wait()
            smD.wait()
            if i + 1 < K:
                smA = pltpu.async_copy(asrs.at[sg.at[i + 1]],
                                       asv[ib ^ 1], ssmA[ib ^ 1])
                smD = pltpu.async_copy(adss.at[dg.at[i + 1]],
                                       adv[ib ^ 1], ssmD[ib ^ 1])
                if sc_prev is not None:
                    sc_prev.wait()
                    sc_prev = None
                gat_next = _gat2(i + 1, ib ^ 1)
            if den_prev[ib] is not None:
                den_prev[ib].wait()
            for kk in range(CH // 16):
                sl = pl.ds(kk * 16, 16)
                a = asv[ib][sl] + adv[ib][sl] + aeg[i, sl]
                a = jnp.where(a >= 0.0, a, 0.2 * a)
                exc[ib][sl] = jnp.exp(a - Mv)
            den_prev[ib] = pltpu.async_copy(exc[ib], dens.at[dg.at[i]],
                                            sden[ib], add=True)
            gat[0].wait()
            gat[1].wait()
            _scale(exc[ib], rb[ib])
            sc_cur = pltpu.async_copy(rb[ib], accs.at[dg.at[i]],
                                      ssc[ib], add=True)
            if sc_prev is not None:
                sc_prev.wait()
            sc_prev = sc_cur
            if i + 1 < K:
                gat = gat_next
        sc_prev.wait()
        for dpd in den_prev:
            if dpd is not None:
                dpd.wait()

    def _pair(t, _):
        _group(t * 2, sb0, db0, aec0)
        _group(t * 2 + 1, sb1, db1, aec1)
        return 0

    lax.fori_loop(0, NCH // K // 2, _pair, 0)
    plsc.subcore_barrier()

    rows = pl.ds(sid * RPT, RPT)

    @pl.when(cid == 0)
    def _():
        pltpu.sync_copy(accs.at[rows], acc_hbm.at[0, rows])
        for t in range(RPT // CH):
            st = pl.ds(sid * RPT + t * CH, CH)
            pltpu.sync_copy(dens.at[st], exc0)
            pltpu.sync_copy(exc0, d0_hbm.at[st])

    @pl.when(cid == 1)
    def _():
        pltpu.sync_copy(accs.at[rows], acc_hbm.at[1, rows])
        for t in range(RPT // CH):
            st = pl.ds(sid * RPT + t * CH, CH)
            pltpu.sync_copy(dens.at[st], exc0)
            pltpu.sync_copy(exc0, d1_hbm.at[st])


@functools.partial(
    pl.kernel,
    out_type=(
        _sds((NC, NP, D)),
        _sds((NP,)),
        _sds((NP,)),
    ),
    mesh=plsc.VectorSubcoreMesh(core_axis_name="c", subcore_axis_name="s"),
    compiler_params=pltpu.CompilerParams(needs_layout_passes=False),
    scratch_types=[
        pltpu.VMEM((D,), _f32),
        pltpu.VMEM((K, CH), jnp.int32),
        pltpu.VMEM((K, CH), jnp.int32),
        pltpu.VMEM((K, CH), jnp.int32),
        pltpu.VMEM((K, CH), jnp.int32),
        pltpu.VMEM((K, CH), _f32),
        pltpu.VMEM((K, CH), _f32),
        pltpu.VMEM((CH,), _f32),
        pltpu.VMEM((CH,), _f32),
        pltpu.VMEM((CH,), _f32),
        pltpu.VMEM((CH,), _f32),
        pltpu.VMEM((CH,), _f32),
        pltpu.VMEM((CH,), _f32),
        pltpu.VMEM((CH, D), _f32),
        pltpu.VMEM((CH, D), _f32),
        pltpu.VMEM_SHARED((NP, D), _f32),
        pltpu.VMEM_SHARED((NP,), _f32),
        pltpu.VMEM_SHARED((NP,), _f32),
        pltpu.VMEM_SHARED((NP,), _f32),
        pltpu.SemaphoreType.DMA,
        pltpu.SemaphoreType.DMA,
        pltpu.SemaphoreType.DMA,
        pltpu.SemaphoreType.DMA,
        pltpu.SemaphoreType.DMA,
        pltpu.SemaphoreType.DMA,
        pltpu.SemaphoreType.DMA,
        pltpu.SemaphoreType.DMA,
        pltpu.SemaphoreType.DMA,
        pltpu.SemaphoreType.DMA,
    ],
)
def _sc_layer(h_hbm, as_hbm, ad_hbm, m_hbm, src_hbm, dst_hbm, ae_hbm,
              acc_hbm, d0_hbm, d1_hbm, *scratch):
    _sc_body(h_hbm, as_hbm, ad_hbm, m_hbm, src_hbm, dst_hbm, ae_hbm,
             acc_hbm, d0_hbm, d1_hbm, *scratch)


def kernel(x, edge_index, edge_attr, W, att_src, att_dst, We, att_e, b,
           gamma, beta):
    x = x.astype(_f32)
    edge_attr = edge_attr.astype(_f32)
    W = W.astype(_f32)
    b = b.astype(_f32)
    src = edge_index[0].astype(jnp.int32).reshape(NW, EPW)
    dst = edge_index[1].astype(jnp.int32).reshape(NW, EPW)

    eaT = edge_attr.T.reshape(3, E // D, D)
    aed, mae = _tc_edge(eaT, We.astype(_f32), att_e.astype(_f32))

    pad_i = jnp.zeros((NW, NP - EPW), jnp.int32)
    srcp = jnp.concatenate([src, pad_i], axis=1).reshape(NW, NCH, CH)
    dstp = jnp.concatenate([dst, pad_i], axis=1).reshape(NW, NCH, CH)
    aedp = jnp.concatenate(
        [aed.reshape(NL, NW, EPW),
         jnp.full((NL, NW, NP - EPW), NEG, _f32)], axis=2,
    ).reshape(NL, NW, NCH, CH)

    xp = jnp.pad(x, ((0, NP - N), (0, 0)))
    h, s, d, m = _tc_in(xp, W[0], att_src[0], att_dst[0], mae[0])

    acc = den = None
    for l in range(NL):
        acc, d0, d1 = _sc_layer(h, s.reshape(NP), d.reshape(NP),
                                m.reshape(D), srcp, dstp, aedp[l])
        den = jnp.stack([d0, d1]).reshape(NC, NP, 1)
        if l < NL - 1:
            h, s, d, m = _tc_mid(acc, den, b[l], W[l + 1],
                                 att_src[l + 1], att_dst[l + 1],
                                 mae[l + 1])

    out = _tc_fin(acc, den, b[NL - 1], gamma.astype(_f32),
                  beta.astype(_f32))
    return out[:N]


# speedup vs baseline: 1.0185x; 1.0185x over previous
import functools

import numpy as np

import jax
import jax.numpy as jnp
from jax import lax
from jax.experimental import pallas as pl
from jax.experimental.pallas import tpu as pltpu
from jax.experimental.pallas import tpu_sc as plsc

N = 10000
NP = 10240
D = 128
E = 320000
NL = 4
NC, NS = 2, 16
NW = NC * NS
EPW = E // NW
CH = 128
K = 8
NCH = NP // CH
RPT = NP // NS
NEG = -1e30

_f32 = jnp.float32

_PERM = np.concatenate([np.arange(0, D, 2), np.arange(1, D, 2)])
_INV = np.argsort(_PERM)


def _sds(shape, dtype=_f32):
    return jax.ShapeDtypeStruct(shape, dtype)


def _tc_edge_body(ea_ref, we_ref, ae_ref, aed_ref, mae_ref):
    rv = jnp.sum(we_ref[...] * ae_ref[...][:, None, :], axis=-1,
                 keepdims=True)
    for l in range(NL):
        acc = None
        for j in range(3):
            coef = rv[l, j:j + 1, :]
            t = ea_ref[j] * coef
            acc = t if acc is None else acc + t
        aed_ref[l] = acc
        mae_ref[l:l + 1, :] = jnp.max(acc, axis=0, keepdims=True)


def _tc_edge(eaT, We, att_e):
    return pl.pallas_call(
        _tc_edge_body,
        out_shape=(_sds((NL, E // D, D)), _sds((NL, D))),
    )(eaT, We, att_e)


_BR = 2048


def _attn_tail(h, as_ref, ad_ref, mae_ref, s_ref, d_ref, m_ref, sm_ref):
    s = jnp.sum(h * as_ref[...][None, :], axis=-1, keepdims=True)
    d = jnp.sum(h * ad_ref[...][None, :], axis=-1, keepdims=True)
    s_ref[...] = s
    d_ref[...] = d
    ms = jnp.max(s)
    md = jnp.max(d)

    @pl.when(pl.program_id(0) == 0)
    def _():
        sm_ref[0] = ms
        sm_ref[1] = md
        m_ref[...] = jnp.zeros((1, D), _f32)

    @pl.when(pl.program_id(0) > 0)
    def _():
        sm_ref[0] = jnp.maximum(sm_ref[0], ms)
        sm_ref[1] = jnp.maximum(sm_ref[1], md)

    @pl.when(pl.program_id(0) == NP // _BR - 1)
    def _():
        M = jnp.maximum(sm_ref[0] + sm_ref[1] + jnp.max(mae_ref[...]), 0.0)
        m_ref[...] = jnp.broadcast_to(M, (1, D))


def _tc_in_body(x_ref, w_ref, as_ref, ad_ref, mae_ref,
                h_ref, s_ref, d_ref, m_ref, sm_ref):
    h = jnp.dot(x_ref[...], w_ref[...], preferred_element_type=_f32)
    h_ref[...] = h
    _attn_tail(h, as_ref, ad_ref, mae_ref, s_ref, d_ref, m_ref, sm_ref)


_NODE_OUT_SPECS = [
    pl.BlockSpec((_BR, D), lambda i: (i, 0)),
    pl.BlockSpec((_BR, 1), lambda i: (i, 0)),
    pl.BlockSpec((_BR, 1), lambda i: (i, 0)),
    pl.BlockSpec((1, D), lambda i: (0, 0)),
]
_NODE_OUT_SHAPE = (_sds((NP, D)), _sds((NP, 1)),
                   _sds((NP, 1)), _sds((1, D)))


def _tc_in(xp, W0, as0, ad0, mae0):
    return pl.pallas_call(
        _tc_in_body,
        grid=(NP // _BR,),
        in_specs=[
            pl.BlockSpec((_BR, D), lambda i: (i, 0)),
            pl.BlockSpec((D, D), lambda i: (0, 0)),
            pl.BlockSpec((D,), lambda i: (0,)),
            pl.BlockSpec((D,), lambda i: (0,)),
            pl.BlockSpec((D,), lambda i: (0,)),
        ],
        out_specs=_NODE_OUT_SPECS,
        out_shape=_NODE_OUT_SHAPE,
        scratch_shapes=[pltpu.SMEM((2,), _f32)],
    )(xp, W0, as0, ad0, mae0)


def _act(acc_ref, den_ref, b_ref):
    den = den_ref[0] + den_ref[1]
    act = (acc_ref[0] + acc_ref[1]) * (1.0 / (den + 1e-16)) \
        + b_ref[...][None, :]
    return jnp.maximum(act, 0.0)


def _tc_mid_body(acc_ref, den_ref, b_ref, w_ref, as_ref, ad_ref, mae_ref,
                 h_ref, s_ref, d_ref, m_ref, sm_ref):
    act = _act(acc_ref, den_ref, b_ref)
    h = jnp.dot(act, w_ref[...], preferred_element_type=_f32)
    h_ref[...] = h
    _attn_tail(h, as_ref, ad_ref, mae_ref, s_ref, d_ref, m_ref, sm_ref)


def _tc_mid(acc, den, bl, Wl, asl, adl, mael):
    return pl.pallas_call(
        _tc_mid_body,
        grid=(NP // _BR,),
        in_specs=[
            pl.BlockSpec((NC, _BR, D), lambda i: (0, i, 0)),
            pl.BlockSpec((NC, _BR, 1), lambda i: (0, i, 0)),
            pl.BlockSpec((D,), lambda i: (0,)),
            pl.BlockSpec((D, D), lambda i: (0, 0)),
            pl.BlockSpec((D,), lambda i: (0,)),
            pl.BlockSpec((D,), lambda i: (0,)),
            pl.BlockSpec((D,), lambda i: (0,)),
        ],
        out_specs=_NODE_OUT_SPECS,
        out_shape=_NODE_OUT_SHAPE,
        scratch_shapes=[pltpu.SMEM((2,), _f32)],
    )(acc, den, bl, Wl, asl, adl, mael)


def _tc_fin_body(acc_ref, den_ref, b_ref, g_ref, be_ref, o_ref):
    act = _act(acc_ref, den_ref, b_ref)
    mu = jnp.mean(act, axis=-1, keepdims=True)
    xc = act - mu
    var = jnp.mean(xc * xc, axis=-1, keepdims=True)
    o_ref[...] = g_ref[...][None, :] * xc * lax.rsqrt(var + 1e-5) \
        + be_ref[...][None, :]


def _tc_fin(acc, den, bl, gamma, beta):
    return pl.pallas_call(
        _tc_fin_body,
        grid=(NP // _BR,),
        in_specs=[
            pl.BlockSpec((NC, _BR, D), lambda i: (0, i, 0)),
            pl.BlockSpec((NC, _BR, 1), lambda i: (0, i, 0)),
            pl.BlockSpec((D,), lambda i: (0,)),
            pl.BlockSpec((D,), lambda i: (0,)),
            pl.BlockSpec((D,), lambda i: (0,)),
        ],
        out_specs=pl.BlockSpec((_BR, D), lambda i: (i, 0)),
        out_shape=_sds((NP, D)),
    )(acc, den, bl, gamma, beta)


DW = D // 2


def _sc_body(h_hbm, as_hbm, ad_hbm, m_hbm, src_hbm, dst_hbm, ae_hbm,
             acc_hbm, d0_hbm, d1_hbm,
             mv, sb0, sb1, db0, db1, aec0, aec1,
             asv0, asv1, adv0, adv1, exc0, exc1,
             rbp0, rbp1,
             accs, dens, asrs, adss,
             srow0, srow1, ssc0, ssc1,
             ssmA0, ssmA1, ssmD0, ssmD1, sden0, sden1):
    cid = lax.axis_index("c")
    sid = lax.axis_index("s")
    wid = cid * NS + sid
    rb = (rbp0, rbp1)
    srow = (srow0, srow1)
    ssc = (ssc0, ssc1)
    asv = (asv0, asv1)
    adv = (adv0, adv1)
    exc = (exc0, exc1)
    ssmA = (ssmA0, ssmA1)
    ssmD = (ssmD0, ssmD1)
    sden = (sden0, sden1)

    pltpu.sync_copy(m_hbm, mv)

    @pl.when(sid == 0)
    def _():
        pltpu.sync_copy(as_hbm, asrs)
        pltpu.sync_copy(ad_hbm, adss)

    zero16 = jnp.zeros((16,), _f32)

    for kk in range(CH // 16):
        exc0[pl.ds(kk * 16, 16)] = zero16

    def _z2(r, _):
        for kk in range(D // 16):
            rbp0[r, pl.ds(kk * 16, 16)] = zero16
        return 0
    lax.fori_loop(0, CH, _z2, 0)

    for t in range(RPT // CH):
        pltpu.sync_copy(rbp0, accs.at[pl.ds(sid * RPT + t * CH, CH)])
        pltpu.sync_copy(exc0, dens.at[pl.ds(sid * RPT + t * CH, CH)])
    plsc.subcore_barrier()

    Mv = mv[pl.ds(0, 16)]

    def _scale(exr, rbr):
        def body(r, _):
            eb = plsc.load_gather(exr, [jnp.full((16,), r, jnp.int32)])
            for kk in range(D // 16):
                sl = pl.ds(kk * 16, 16)
                rbr[r, sl] = rbr[r, sl] * eb
            return 0
        lax.fori_loop(0, CH, body, 0, unroll=8)

    def _group(g, sg, dg, aeg):
        pltpu.sync_copy(src_hbm.at[wid, pl.ds(g * K, K)], sg)
        pltpu.sync_copy(dst_hbm.at[wid, pl.ds(g * K, K)], dg)
        pltpu.sync_copy(ae_hbm.at[wid, pl.ds(g * K, K)], aeg)
        gat = pltpu.async_copy(h_hbm.at[sg.at[0]], rb[0], srow[0])
        smA = pltpu.async_copy(asrs.at[sg.at[0]], asv[0], ssmA[0])
        smD = pltpu.async_copy(adss.at[dg.at[0]], adv[0], ssmD[0])
        sc_prev = None
        den_prev = [None, None]
        for i in range(K):
            ib = i & 1
            smA.wait()
            smD.wait()
            if i + 1 < K:
                smA = pltpu.async_copy(asrs.at[sg.at[i + 1]],
                                       asv[ib ^ 1], ssmA[ib ^ 1])
                smD = pltpu.async_copy(adss.at[dg.at[i + 1]],
                                       adv[ib ^ 1], ssmD[ib ^ 1])
            if den_prev[ib] is not None:
                den_prev[ib].wait()
            for kk in range(CH // 16):
                sl = pl.ds(kk * 16, 16)
                a = asv[ib][sl] + adv[ib][sl] + aeg[i, sl]
                a = jnp.where(a >= 0.0, a, 0.2 * a)
                exc[ib][sl] = jnp.exp(a - Mv)
            den_prev[ib] = pltpu.async_copy(exc[ib], dens.at[dg.at[i]],
                                            sden[ib], add=True)
            if i + 1 < K:
                if sc_prev is not None:
                    sc_prev.wait()
                    sc_prev = None
                gat_next = pltpu.async_copy(h_hbm.at[sg.at[i + 1]],
                                            rb[ib ^ 1], srow[ib ^ 1])
            gat.wait()
            _scale(exc[ib], rb[ib])
            sc_cur = pltpu.async_copy(rb[ib], accs.at[dg.at[i]],
                                      ssc[ib], add=True)
            if sc_prev is not None:
                sc_prev.wait()
            sc_prev = sc_cur
            if i + 1 < K:
                gat = gat_next
        sc_prev.wait()
        for dpd in den_prev:
            if dpd is not None:
                dpd.wait()

    def _pair(t, _):
        _group(t * 2, sb0, db0, aec0)
        _group(t * 2 + 1, sb1, db1, aec1)
        return 0

    lax.fori_loop(0, NCH // K // 2, _pair, 0)
    plsc.subcore_barrier()

    rows = pl.ds(sid * RPT, RPT)

    @pl.when(cid == 0)
    def _():
        pltpu.sync_copy(accs.at[rows], acc_hbm.at[0, rows])
        for t in range(RPT // CH):
            st = pl.ds(sid * RPT + t * CH, CH)
            pltpu.sync_copy(dens.at[st], exc0)
            pltpu.sync_copy(exc0, d0_hbm.at[st])

    @pl.when(cid == 1)
    def _():
        pltpu.sync_copy(accs.at[rows], acc_hbm.at[1, rows])
        for t in range(RPT // CH):
            st = pl.ds(sid * RPT + t * CH, CH)
            pltpu.sync_copy(dens.at[st], exc0)
            pltpu.sync_copy(exc0, d1_hbm.at[st])


@functools.partial(
    pl.kernel,
    out_type=(
        _sds((NC, NP, D)),
        _sds((NP,)),
        _sds((NP,)),
    ),
    mesh=plsc.VectorSubcoreMesh(core_axis_name="c", subcore_axis_name="s"),
    compiler_params=pltpu.CompilerParams(needs_layout_passes=False),
    scratch_types=[
        pltpu.VMEM((D,), _f32),
        pltpu.VMEM((K, CH), jnp.int32),
        pltpu.VMEM((K, CH), jnp.int32),
        pltpu.VMEM((K, CH), jnp.int32),
        pltpu.VMEM((K, CH), jnp.int32),
        pltpu.VMEM((K, CH), _f32),
        pltpu.VMEM((K, CH), _f32),
        pltpu.VMEM((CH,), _f32),
        pltpu.VMEM((CH,), _f32),
        pltpu.VMEM((CH,), _f32),
        pltpu.VMEM((CH,), _f32),
        pltpu.VMEM((CH,), _f32),
        pltpu.VMEM((CH,), _f32),
        pltpu.VMEM((CH, D), _f32),
        pltpu.VMEM((CH, D), _f32),
        pltpu.VMEM_SHARED((NP, D), _f32),
        pltpu.VMEM_SHARED((NP,), _f32),
        pltpu.VMEM_SHARED((NP,), _f32),
        pltpu.VMEM_SHARED((NP,), _f32),
        pltpu.SemaphoreType.DMA,
        pltpu.SemaphoreType.DMA,
        pltpu.SemaphoreType.DMA,
        pltpu.SemaphoreType.DMA,
        pltpu.SemaphoreType.DMA,
        pltpu.SemaphoreType.DMA,
        pltpu.SemaphoreType.DMA,
        pltpu.SemaphoreType.DMA,
        pltpu.SemaphoreType.DMA,
        pltpu.SemaphoreType.DMA,
    ],
)
def _sc_layer(h_hbm, as_hbm, ad_hbm, m_hbm, src_hbm, dst_hbm, ae_hbm,
              acc_hbm, d0_hbm, d1_hbm, *scratch):
    _sc_body(h_hbm, as_hbm, ad_hbm, m_hbm, src_hbm, dst_hbm, ae_hbm,
             acc_hbm, d0_hbm, d1_hbm, *scratch)


def kernel(x, edge_index, edge_attr, W, att_src, att_dst, We, att_e, b,
           gamma, beta):
    x = x.astype(_f32)
    edge_attr = edge_attr.astype(_f32)
    W = W.astype(_f32)
    b = b.astype(_f32)
    src = edge_index[0].astype(jnp.int32).reshape(NW, EPW)
    dst = edge_index[1].astype(jnp.int32).reshape(NW, EPW)

    eaT = edge_attr.T.reshape(3, E // D, D)
    aed, mae = _tc_edge(eaT, We.astype(_f32), att_e.astype(_f32))

    pad_i = jnp.zeros((NW, NP - EPW), jnp.int32)
    srcp = jnp.concatenate([src, pad_i], axis=1).reshape(NW, NCH, CH)
    dstp = jnp.concatenate([dst, pad_i], axis=1).reshape(NW, NCH, CH)
    aedp = jnp.concatenate(
        [aed.reshape(NL, NW, EPW),
         jnp.full((NL, NW, NP - EPW), NEG, _f32)], axis=2,
    ).reshape(NL, NW, NCH, CH)

    xp = jnp.pad(x, ((0, NP - N), (0, 0)))
    h, s, d, m = _tc_in(xp, W[0], att_src[0], att_dst[0], mae[0])

    acc = den = None
    for l in range(NL):
        acc, d0, d1 = _sc_layer(h, s.reshape(NP), d.reshape(NP),
                                m.reshape(D), srcp, dstp, aedp[l])
        den = jnp.stack([d0, d1]).reshape(NC, NP, 1)
        if l < NL - 1:
            h, s, d, m = _tc_mid(acc, den, b[l], W[l + 1],
                                 att_src[l + 1], att_dst[l + 1],
                                 mae[l + 1])

    out = _tc_fin(acc, den, b[NL - 1], gamma.astype(_f32),
                  beta.astype(_f32))
    return out[:N]
